# Initial kernel scaffold; baseline (speedup 1.0000x reference)
import jax
import jax.numpy as jnp
from jax.experimental import pallas as pl


def kernel(x, edge_index, W_lift, b_lift, W_rel, b_rel, W_root, W_proj, b_proj):
    raise NotImplementedError("write your pallas kernel here")


# speedup vs baseline: 5.2332x; 5.2332x over previous
import functools

import jax
import jax.numpy as jnp
from jax import lax
from jax.experimental import pallas as pl
from jax.experimental.pallas import tpu as pltpu
from jax.experimental.pallas import tpu_sc as plsc

_N = 10000
_E = 320000
_D = 128

_NC = 2
_NS = 16
_EP_CORE = _E // _NC
_EP_TILE = _EP_CORE // _NS
_CHUNK = 80
_NCHUNK = _EP_TILE // _CHUNK
_NPAD = 10240
_ROWS_PT = _NPAD // _NS


def _lift_body(x_ref, w_ref, b_ref, o_ref):
    o_ref[...] = (
        jnp.dot(x_ref[...], w_ref[...], preferred_element_type=jnp.float32)
        + b_ref[...]
    )


def _lift(x, w, b):
    blk = 1000
    return pl.pallas_call(
        _lift_body,
        grid=(_N // blk,),
        in_specs=[
            pl.BlockSpec((blk, _D), lambda i: (i, 0)),
            pl.BlockSpec((_D, _D), lambda i: (0, 0)),
            pl.BlockSpec((1, _D), lambda i: (0, 0)),
        ],
        out_specs=pl.BlockSpec((blk, _D), lambda i: (i, 0)),
        out_shape=jax.ShapeDtypeStruct((_N, _D), jnp.float32),
    )(x, w, b.reshape(1, _D))


def _seg_body(h_hbm, src_hbm, dst_hbm, zeros_hbm, out_hbm,
              is0, id0, r0, agg, gsem):
    c = lax.axis_index("c")
    s = lax.axis_index("s")
    ebase = c * _EP_CORE + s * _EP_TILE

    pltpu.sync_copy(zeros_hbm, agg.at[pl.ds(s * _ROWS_PT, _ROWS_PT)])
    plsc.subcore_barrier()

    def body(ci, _):
        off = ebase + ci * _CHUNK
        pltpu.sync_copy(src_hbm.at[pl.ds(off, _CHUNK)], is0)
        pltpu.sync_copy(dst_hbm.at[pl.ds(off, _CHUNK)], id0)
        pltpu.async_copy(h_hbm.at[is0], r0, gsem).wait()
        pltpu.sync_copy(r0, agg.at[id0], add=True)
        return 0

    lax.fori_loop(0, _NCHUNK, body, 0)
    plsc.subcore_barrier()

    pltpu.sync_copy(
        agg.at[pl.ds(s * _ROWS_PT, _ROWS_PT)],
        out_hbm.at[c, pl.ds(s * _ROWS_PT, _ROWS_PT)],
    )


_seg = functools.partial(
    pl.kernel,
    out_type=jax.ShapeDtypeStruct((_NC, _NPAD, _D), jnp.float32),
    mesh=plsc.VectorSubcoreMesh(core_axis_name="c", subcore_axis_name="s"),
    scratch_types=[
        pltpu.VMEM((_CHUNK,), jnp.int32),
        pltpu.VMEM((_CHUNK,), jnp.int32),
        pltpu.VMEM((_CHUNK, _D), jnp.float32),
        pltpu.VMEM_SHARED((_NPAD, _D), jnp.float32),
        pltpu.SemaphoreType.DMA,
    ],
)(_seg_body)


def _tail_body(p0_ref, p1_ref, h_ref, wrel_ref, brel_ref, wroot_ref,
               wproj_ref, bproj_ref, o_ref):
    agg = p0_ref[...] + p1_ref[...]
    t = jnp.tanh(
        jnp.dot(agg, wrel_ref[...], preferred_element_type=jnp.float32)
        + brel_ref[...]
        + jnp.dot(h_ref[...], wroot_ref[...], preferred_element_type=jnp.float32)
    )
    o_ref[...] = (
        jnp.dot(t, wproj_ref[...], preferred_element_type=jnp.float32)
        + bproj_ref[...]
    )


def _tail(p0, p1, h, w_rel, b_rel, w_root, w_proj, b_proj):
    blk = 1000
    full = pl.BlockSpec((_D, _D), lambda i: (0, 0))
    bias = pl.BlockSpec((1, _D), lambda i: (0, 0))
    row = pl.BlockSpec((blk, _D), lambda i: (i, 0))
    return pl.pallas_call(
        _tail_body,
        grid=(_N // blk,),
        in_specs=[row, row, row, full, bias, full, full, bias],
        out_specs=row,
        out_shape=jax.ShapeDtypeStruct((_N, _D), jnp.float32),
    )(p0, p1, h, w_rel, b_rel.reshape(1, _D), w_root, w_proj,
      b_proj.reshape(1, _D))


def kernel(x, edge_index, W_lift, b_lift, W_rel, b_rel, W_root, W_proj,
           b_proj):
    src = edge_index[0].astype(jnp.int32)
    dst = edge_index[1].astype(jnp.int32)
    h = _lift(x, W_lift, b_lift)
    zeros = jnp.zeros((_ROWS_PT, _D), jnp.float32)
    partials = _seg(h, src, dst, zeros)
    return _tail(partials[0], partials[1], h, W_rel, b_rel, W_root, W_proj,
                 b_proj)


# speedup vs baseline: 5.9844x; 1.1435x over previous
import functools

import jax
import jax.numpy as jnp
from jax import lax
from jax.experimental import pallas as pl
from jax.experimental.pallas import tpu as pltpu
from jax.experimental.pallas import tpu_sc as plsc

_N = 10000
_E = 320000
_D = 128

_NC = 2
_NS = 16
_EP_CORE = _E // _NC
_EP_TILE = _EP_CORE // _NS
_CHUNK = 40
_NCHUNK = _EP_TILE // _CHUNK
_NPAD = 10240
_ROWS_PT = _NPAD // _NS

_NBUF = 5
_NIB = 2 * _NBUF


def _lift_body(x_ref, w_ref, b_ref, o_ref):
    o_ref[...] = (
        jnp.dot(x_ref[...], w_ref[...], preferred_element_type=jnp.float32)
        + b_ref[...]
    )


def _lift(x, w, b):
    blk = 1000
    return pl.pallas_call(
        _lift_body,
        grid=(_N // blk,),
        in_specs=[
            pl.BlockSpec((blk, _D), lambda i: (i, 0)),
            pl.BlockSpec((_D, _D), lambda i: (0, 0)),
            pl.BlockSpec((1, _D), lambda i: (0, 0)),
        ],
        out_specs=pl.BlockSpec((blk, _D), lambda i: (i, 0)),
        out_shape=jax.ShapeDtypeStruct((_N, _D), jnp.float32),
    )(x, w, b.reshape(1, _D))


def _seg_body(h_hbm, eidx_hbm, zeros_hbm, out_hbm, ibufs, rows, agg,
              iqs, gsems, ssems):
    c = lax.axis_index("c")
    s = lax.axis_index("s")
    wid = c * _NS + s

    pltpu.sync_copy(zeros_hbm, agg.at[pl.ds(s * _ROWS_PT, _ROWS_PT)])
    plsc.subcore_barrier()

    def ifetch(ci, q):
        pltpu.async_copy(eidx_hbm.at[wid, ci], ibufs[q], iqs[q])

    def wait_ifetch(q):
        pltpu.make_async_copy(eidx_hbm.at[0, 0], ibufs[q], iqs[q]).wait()

    def gather(q, b):
        pltpu.async_copy(h_hbm.at[ibufs[q].at[0]], rows[b], gsems[b])

    def wait_gather(b):
        pltpu.make_async_copy(h_hbm.at[pl.ds(0, _CHUNK)], rows[b],
                              gsems[b]).wait()

    def scatter(q, b):
        pltpu.async_copy(rows[b], agg.at[ibufs[q].at[1]], ssems[b],
                         add=True)

    def wait_scatter(b):
        pltpu.make_async_copy(h_hbm.at[pl.ds(0, _CHUNK)], rows[b],
                              ssems[b]).wait()

    for q in range(_NBUF):
        ifetch(q, q)
    for n in range(_NIB):
        b = n % _NBUF
        if n >= _NBUF:
            wait_scatter(b)
        ifetch(n + _NBUF, (n + _NBUF) % _NIB)
        if n >= 1:
            pb = (n - 1) % _NBUF
            wait_gather(pb)
            scatter((n - 1) % _NIB, pb)
        wait_ifetch(n % _NIB)
        gather(n % _NIB, b)

    @pl.loop(1, _NCHUNK // _NIB)
    def _steady(t):
        for jj in range(_NIB):
            n = t * _NIB + jj
            b = jj % _NBUF
            wait_scatter(b)
            nxt = n + _NBUF

            @pl.when(nxt < _NCHUNK)
            def _():
                ifetch(nxt, (jj + _NBUF) % _NIB)

            pb = (jj - 1) % _NBUF
            wait_gather(pb)
            scatter((jj - 1) % _NIB, pb)
            wait_ifetch(jj)
            gather(jj, b)

    lb = (_NCHUNK - 1) % _NBUF
    wait_gather(lb)
    scatter((_NCHUNK - 1) % _NIB, lb)
    for b in range(_NBUF):
        wait_scatter(b)

    plsc.subcore_barrier()

    pltpu.sync_copy(
        agg.at[pl.ds(s * _ROWS_PT, _ROWS_PT)],
        out_hbm.at[c, pl.ds(s * _ROWS_PT, _ROWS_PT)],
    )


_seg = functools.partial(
    pl.kernel,
    out_type=jax.ShapeDtypeStruct((_NC, _NPAD, _D), jnp.float32),
    mesh=plsc.VectorSubcoreMesh(core_axis_name="c", subcore_axis_name="s"),
    scratch_types=[
        [pltpu.VMEM((2, _CHUNK), jnp.int32) for _ in range(_NIB)],
        [pltpu.VMEM((_CHUNK, _D), jnp.float32) for _ in range(_NBUF)],
        pltpu.VMEM_SHARED((_NPAD, _D), jnp.float32),
        [pltpu.SemaphoreType.DMA for _ in range(_NIB)],
        [pltpu.SemaphoreType.DMA for _ in range(_NBUF)],
        [pltpu.SemaphoreType.DMA for _ in range(_NBUF)],
    ],
)(_seg_body)


def _tail_body(p0_ref, p1_ref, h_ref, wrel_ref, brel_ref, wroot_ref,
               wproj_ref, bproj_ref, o_ref):
    agg = p0_ref[...] + p1_ref[...]
    t = jnp.tanh(
        jnp.dot(agg, wrel_ref[...], preferred_element_type=jnp.float32)
        + brel_ref[...]
        + jnp.dot(h_ref[...], wroot_ref[...], preferred_element_type=jnp.float32)
    )
    o_ref[...] = (
        jnp.dot(t, wproj_ref[...], preferred_element_type=jnp.float32)
        + bproj_ref[...]
    )


def _tail(p0, p1, h, w_rel, b_rel, w_root, w_proj, b_proj):
    blk = 1000
    full = pl.BlockSpec((_D, _D), lambda i: (0, 0))
    bias = pl.BlockSpec((1, _D), lambda i: (0, 0))
    row = pl.BlockSpec((blk, _D), lambda i: (i, 0))
    return pl.pallas_call(
        _tail_body,
        grid=(_N // blk,),
        in_specs=[row, row, row, full, bias, full, full, bias],
        out_specs=row,
        out_shape=jax.ShapeDtypeStruct((_N, _D), jnp.float32),
    )(p0, p1, h, w_rel, b_rel.reshape(1, _D), w_root, w_proj,
      b_proj.reshape(1, _D))


def kernel(x, edge_index, W_lift, b_lift, W_rel, b_rel, W_root, W_proj,
           b_proj):
    nw = _NC * _NS
    src = edge_index[0].astype(jnp.int32).reshape(nw, _NCHUNK, 1, _CHUNK)
    dst = edge_index[1].astype(jnp.int32).reshape(nw, _NCHUNK, 1, _CHUNK)
    eidx = jnp.concatenate([src, dst], axis=2)
    h = _lift(x, W_lift, b_lift)
    zeros = jnp.zeros((_ROWS_PT, _D), jnp.float32)
    partials = _seg(h, eidx, zeros)
    return _tail(partials[0], partials[1], h, W_rel, b_rel, W_root, W_proj,
                 b_proj)


# speedup vs baseline: 6.0722x; 1.0147x over previous
import functools

import jax
import jax.numpy as jnp
from jax import lax
from jax.experimental import pallas as pl
from jax.experimental.pallas import tpu as pltpu
from jax.experimental.pallas import tpu_sc as plsc

_N = 10000
_E = 320000
_D = 128

_NC = 2
_NS = 16
_EP_CORE = _E // _NC
_EP_TILE = _EP_CORE // _NS
_CHUNK = 80
_NCHUNK = 126
_NPAD = 10240
_ROWS_PT = _NPAD // _NS
_DUMP = _NPAD - 1

_NBUF = 3
_NIB = 2 * _NBUF


def _lift_body(x_ref, w_ref, b_ref, o_ref):
    o_ref[...] = (
        jnp.dot(x_ref[...], w_ref[...], preferred_element_type=jnp.float32)
        + b_ref[...]
    )


def _lift(x, w, b):
    blk = 1000
    return pl.pallas_call(
        _lift_body,
        grid=(_N // blk,),
        in_specs=[
            pl.BlockSpec((blk, _D), lambda i: (i, 0)),
            pl.BlockSpec((_D, _D), lambda i: (0, 0)),
            pl.BlockSpec((1, _D), lambda i: (0, 0)),
        ],
        out_specs=pl.BlockSpec((blk, _D), lambda i: (i, 0)),
        out_shape=jax.ShapeDtypeStruct((_N, _D), jnp.float32),
    )(x, w, b.reshape(1, _D))


def _seg_body(h_hbm, eidx_hbm, zeros_hbm, out_hbm, ibufs, rows, agg,
              iqs, gsems, ssems):
    c = lax.axis_index("c")
    s = lax.axis_index("s")
    wid = c * _NS + s

    pltpu.sync_copy(zeros_hbm, agg.at[pl.ds(s * _ROWS_PT, _ROWS_PT)])
    plsc.subcore_barrier()

    def ifetch(ci, q):
        pltpu.async_copy(eidx_hbm.at[wid, ci], ibufs[q], iqs[q])

    def wait_ifetch(q):
        pltpu.make_async_copy(eidx_hbm.at[0, 0], ibufs[q], iqs[q]).wait()

    def gather(q, b):
        pltpu.async_copy(h_hbm.at[ibufs[q].at[0]], rows[b], gsems[b])

    def wait_gather(b):
        pltpu.make_async_copy(h_hbm.at[pl.ds(0, _CHUNK)], rows[b],
                              gsems[b]).wait()

    def scatter(q, b):
        pltpu.async_copy(rows[b], agg.at[ibufs[q].at[1]], ssems[b],
                         add=True)

    def wait_scatter(b):
        pltpu.make_async_copy(h_hbm.at[pl.ds(0, _CHUNK)], rows[b],
                              ssems[b]).wait()

    for q in range(_NBUF):
        ifetch(q, q)
    for n in range(_NIB):
        b = n % _NBUF
        if n >= _NBUF:
            wait_scatter(b)
        ifetch(n + _NBUF, (n + _NBUF) % _NIB)
        if n >= 1:
            pb = (n - 1) % _NBUF
            wait_gather(pb)
            scatter((n - 1) % _NIB, pb)
        wait_ifetch(n % _NIB)
        gather(n % _NIB, b)

    @pl.loop(1, _NCHUNK // _NIB)
    def _steady(t):
        for jj in range(_NIB):
            n = t * _NIB + jj
            b = jj % _NBUF
            wait_scatter(b)
            nxt = n + _NBUF

            @pl.when(nxt < _NCHUNK)
            def _():
                ifetch(nxt, (jj + _NBUF) % _NIB)

            pb = (jj - 1) % _NBUF
            wait_gather(pb)
            scatter((jj - 1) % _NIB, pb)
            wait_ifetch(jj)
            gather(jj, b)

    lb = (_NCHUNK - 1) % _NBUF
    wait_gather(lb)
    scatter((_NCHUNK - 1) % _NIB, lb)
    for b in range(_NBUF):
        wait_scatter(b)

    plsc.subcore_barrier()

    pltpu.sync_copy(
        agg.at[pl.ds(s * _ROWS_PT, _ROWS_PT)],
        out_hbm.at[c, pl.ds(s * _ROWS_PT, _ROWS_PT)],
    )


_seg = functools.partial(
    pl.kernel,
    out_type=jax.ShapeDtypeStruct((_NC, _NPAD, _D), jnp.float32),
    mesh=plsc.VectorSubcoreMesh(core_axis_name="c", subcore_axis_name="s"),
    scratch_types=[
        [pltpu.VMEM((2, _CHUNK), jnp.int32) for _ in range(_NIB)],
        [pltpu.VMEM((_CHUNK, _D), jnp.float32) for _ in range(_NBUF)],
        pltpu.VMEM_SHARED((_NPAD, _D), jnp.float32),
        [pltpu.SemaphoreType.DMA for _ in range(_NIB)],
        [pltpu.SemaphoreType.DMA for _ in range(_NBUF)],
        [pltpu.SemaphoreType.DMA for _ in range(_NBUF)],
    ],
)(_seg_body)


def _tail_body(p0_ref, p1_ref, h_ref, wrel_ref, brel_ref, wroot_ref,
               wproj_ref, bproj_ref, o_ref):
    agg = p0_ref[...] + p1_ref[...]
    t = jnp.tanh(
        jnp.dot(agg, wrel_ref[...], preferred_element_type=jnp.float32)
        + brel_ref[...]
        + jnp.dot(h_ref[...], wroot_ref[...], preferred_element_type=jnp.float32)
    )
    o_ref[...] = (
        jnp.dot(t, wproj_ref[...], preferred_element_type=jnp.float32)
        + bproj_ref[...]
    )


def _tail(p0, p1, h, w_rel, b_rel, w_root, w_proj, b_proj):
    blk = 1000
    full = pl.BlockSpec((_D, _D), lambda i: (0, 0))
    bias = pl.BlockSpec((1, _D), lambda i: (0, 0))
    row = pl.BlockSpec((blk, _D), lambda i: (i, 0))
    return pl.pallas_call(
        _tail_body,
        grid=(_N // blk,),
        in_specs=[row, row, row, full, bias, full, full, bias],
        out_specs=row,
        out_shape=jax.ShapeDtypeStruct((_N, _D), jnp.float32),
    )(p0, p1, h, w_rel, b_rel.reshape(1, _D), w_root, w_proj,
      b_proj.reshape(1, _D))


def kernel(x, edge_index, W_lift, b_lift, W_rel, b_rel, W_root, W_proj,
           b_proj):
    nw = _NC * _NS
    src = edge_index[0].astype(jnp.int32).reshape(nw, _NCHUNK - 1, 1, _CHUNK)
    dst = edge_index[1].astype(jnp.int32).reshape(nw, _NCHUNK - 1, 1, _CHUNK)
    pad_src = jnp.zeros((nw, 1, 1, _CHUNK), jnp.int32)
    pad_dst = jnp.full((nw, 1, 1, _CHUNK), _DUMP, jnp.int32)
    eidx = jnp.concatenate([
        jnp.concatenate([src, pad_src], axis=1),
        jnp.concatenate([dst, pad_dst], axis=1),
    ], axis=2)
    h = _lift(x, W_lift, b_lift)
    zeros = jnp.zeros((_ROWS_PT, _D), jnp.float32)
    partials = _seg(h, eidx, zeros)
    return _tail(partials[0], partials[1], h, W_rel, b_rel, W_root, W_proj,
                 b_proj)
